# baseline (speedup 1.0000x reference)
import functools

import jax
import jax.numpy as jnp
from jax.experimental import pallas as pl
from jax.experimental.pallas import tpu as pltpu

_Q = 1024
_D = 16
_K = 5
_C = 2048


def _knn_body(x2_ref, xt_ref, y_ref, out_ref, s_ref, *, nsteps):
    j = pl.program_id(0)

    @pl.when(j == 0)
    def _init():
        s_ref[...] = jnp.full((_Q, 8), jnp.inf, dtype=jnp.float32)

    xq = x2_ref[...]
    xt = xt_ref[...]
    y = y_ref[0]

    cross = jax.lax.dot_general(
        xq, xt, dimension_numbers=(((1,), (1,)), ((), ())),
        preferred_element_type=jnp.float32)
    tsq = jnp.sum(xt * xt, axis=1)[None, :]
    xsq = jnp.sum(xq * xq, axis=1, keepdims=True)
    d2 = xsq - 2.0 * cross + tsq

    ki = jax.lax.bitcast_convert_type(d2, jnp.int32)
    ki = jnp.bitwise_or(jnp.bitwise_and(ki, jnp.int32(-2)), y)
    keys = jax.lax.bitcast_convert_type(ki, jnp.float32)

    h = _C // 2
    a, b = keys[:, :h], keys[:, h:]
    m1 = jnp.minimum(a, b)
    m2 = jnp.maximum(a, b)
    while h > 128:
        h //= 2
        a1, b1 = m1[:, :h], m1[:, h:]
        a2, b2 = m2[:, :h], m2[:, h:]
        m2 = jnp.minimum(jnp.maximum(a1, b1), jnp.minimum(a2, b2))
        m1 = jnp.minimum(a1, b1)
    cand = jnp.concatenate([m1, m2], axis=1)

    inf = jnp.float32(jnp.inf)
    svals = s_ref[...]
    news = []
    for i in range(_K):
        mc = jnp.min(cand, axis=1, keepdims=True)
        ms = jnp.min(svals, axis=1, keepdims=True)
        m = jnp.minimum(mc, ms)
        news.append(m)
        if i < _K - 1:
            cand = jnp.where(cand == m, inf, cand)
        svals = jnp.where(svals == m, inf, svals)

    top5 = jnp.concatenate(news, axis=1)
    s_ref[...] = jnp.concatenate(
        [top5, jnp.full((_Q, 3), jnp.inf, dtype=jnp.float32)], axis=1)

    @pl.when(j == nsteps - 1)
    def _finish():
        bits = jnp.bitwise_and(
            jax.lax.bitcast_convert_type(top5, jnp.int32), jnp.int32(1))
        votes = jnp.sum(bits, axis=1, keepdims=True)
        out_ref[...] = (votes > _K // 2).astype(jnp.float32)


@jax.jit
def kernel(x, X_train, y_train):
    n = X_train.shape[0]
    nc = (n + _C - 1) // _C
    npad = nc * _C - n
    Xp = jnp.pad(X_train, ((0, npad), (0, 0)), constant_values=1e15)
    yp = jnp.pad(y_train, (0, npad)).reshape(nc, 1, _C)

    out = pl.pallas_call(
        functools.partial(_knn_body, nsteps=nc),
        grid=(nc,),
        in_specs=[
            pl.BlockSpec((_Q, _D), lambda j: (0, 0)),
            pl.BlockSpec((_C, _D), lambda j: (j, 0)),
            pl.BlockSpec((1, 1, _C), lambda j: (j, 0, 0)),
        ],
        out_specs=pl.BlockSpec((_Q, 1), lambda j: (0, 0)),
        out_shape=jax.ShapeDtypeStruct((_Q, 1), jnp.float32),
        scratch_shapes=[pltpu.VMEM((_Q, 8), jnp.float32)],
    )(x.reshape(_Q, _D), Xp, yp)
    return out


# speedup vs baseline: 1.2973x; 1.2973x over previous
import functools

import jax
import jax.numpy as jnp
from jax.experimental import pallas as pl
from jax.experimental.pallas import tpu as pltpu

_Q = 1024
_D = 16
_K = 5
_C = 2048


def _knn_body(x2_ref, xt_ref, y_ref, tq_ref, out_ref,
              s1_ref, s2_ref, s3_ref, s4_ref, *, nsteps):
    j = pl.program_id(0)
    inf = jnp.float32(jnp.inf)

    @pl.when(j == 0)
    def _init():
        full = jnp.full((_Q, 128), jnp.inf, dtype=jnp.float32)
        s1_ref[...] = full
        s2_ref[...] = full
        s3_ref[...] = full
        s4_ref[...] = full

    x2 = x2_ref[...]
    xt = xt_ref[...]
    y = y_ref[0]
    tq = tq_ref[0]

    cross2 = jax.lax.dot_general(
        x2, xt, dimension_numbers=(((1,), (1,)), ((), ())),
        preferred_element_type=jnp.float32)
    xsq = 0.25 * jnp.sum(x2 * x2, axis=1, keepdims=True)
    d2 = (cross2 + xsq) + tq

    ki = jax.lax.bitcast_convert_type(d2, jnp.int32)
    ki = jnp.bitwise_or(jnp.bitwise_and(ki, jnp.int32(-2)), y)
    keys = jax.lax.bitcast_convert_type(ki, jnp.float32)

    h = _C // 2
    a, b = keys[:, :h], keys[:, h:]
    m1 = jnp.minimum(a, b)
    m2 = jnp.maximum(a, b)
    while h > 128:
        h //= 2
        a1, b1 = m1[:, :h], m1[:, h:]
        a2, b2 = m2[:, :h], m2[:, h:]
        m2 = jnp.minimum(jnp.maximum(a1, b1), jnp.minimum(a2, b2))
        m1 = jnp.minimum(a1, b1)

    a1, a2, a3, a4 = s1_ref[...], s2_ref[...], s3_ref[...], s4_ref[...]
    c1 = jnp.minimum(a1, m1)
    c2 = jnp.minimum(jnp.minimum(a2, jnp.maximum(a1, m1)), m2)
    c3 = jnp.minimum(a3, jnp.minimum(jnp.maximum(a2, m1), jnp.maximum(a1, m2)))
    c4 = jnp.minimum(a4, jnp.minimum(jnp.maximum(a3, m1), jnp.maximum(a2, m2)))
    s1_ref[...] = c1
    s2_ref[...] = c2
    s3_ref[...] = c3
    s4_ref[...] = c4

    @pl.when(j == nsteps - 1)
    def _finish():
        cand = jnp.concatenate([c1, c2, c3, c4], axis=1)
        news = []
        for i in range(_K):
            m = jnp.min(cand, axis=1, keepdims=True)
            news.append(m)
            if i < _K - 1:
                cand = jnp.where(cand == m, inf, cand)
        top5 = jnp.concatenate(news, axis=1)
        bits = jnp.bitwise_and(
            jax.lax.bitcast_convert_type(top5, jnp.int32), jnp.int32(1))
        votes = jnp.sum(bits, axis=1, keepdims=True)
        out_ref[...] = (votes > _K // 2).astype(jnp.float32)


@jax.jit
def kernel(x, X_train, y_train):
    n = X_train.shape[0]
    nc = (n + _C - 1) // _C
    npad = nc * _C - n
    Xp = jnp.pad(X_train, ((0, npad), (0, 0)), constant_values=1e15)
    yp = jnp.pad(y_train, (0, npad)).reshape(nc, 1, _C)
    tqp = jnp.sum(Xp * Xp, axis=1).reshape(nc, 1, _C)
    x2 = x.reshape(_Q, _D) * jnp.float32(-2.0)

    out = pl.pallas_call(
        functools.partial(_knn_body, nsteps=nc),
        grid=(nc,),
        in_specs=[
            pl.BlockSpec((_Q, _D), lambda j: (0, 0)),
            pl.BlockSpec((_C, _D), lambda j: (j, 0)),
            pl.BlockSpec((1, 1, _C), lambda j: (j, 0, 0)),
            pl.BlockSpec((1, 1, _C), lambda j: (j, 0, 0)),
        ],
        out_specs=pl.BlockSpec((_Q, 1), lambda j: (0, 0)),
        out_shape=jax.ShapeDtypeStruct((_Q, 1), jnp.float32),
        scratch_shapes=[pltpu.VMEM((_Q, 128), jnp.float32)] * 4,
    )(x2, Xp, yp, tqp)
    return out


# speedup vs baseline: 1.3639x; 1.0513x over previous
import functools

import jax
import jax.numpy as jnp
from jax.experimental import pallas as pl
from jax.experimental.pallas import tpu as pltpu

_Q = 1024
_D = 16
_K = 5
_C = 4096


def _knn_body(x2_ref, xt_ref, y_ref, tq_ref, out_ref,
              s1_ref, s2_ref, s3_ref, s4_ref, *, nsteps):
    j = pl.program_id(0)
    inf = jnp.float32(jnp.inf)

    @pl.when(j == 0)
    def _init():
        full = jnp.full((_Q, 128), jnp.inf, dtype=jnp.float32)
        s1_ref[...] = full
        s2_ref[...] = full
        s3_ref[...] = full
        s4_ref[...] = full

    x2 = x2_ref[...]
    xt = xt_ref[...]
    y = y_ref[0]
    tq = tq_ref[0]

    cross2 = jax.lax.dot_general(
        x2, xt, dimension_numbers=(((1,), (1,)), ((), ())),
        preferred_element_type=jnp.float32)
    xsq = 0.25 * jnp.sum(x2 * x2, axis=1, keepdims=True)
    d2 = (cross2 + xsq) + tq

    ki = jax.lax.bitcast_convert_type(d2, jnp.int32)
    ki = jnp.bitwise_or(jnp.bitwise_and(ki, jnp.int32(-2)), y)
    keys = jax.lax.bitcast_convert_type(ki, jnp.float32)

    h = _C // 2
    a, b = keys[:, :h], keys[:, h:]
    m1 = jnp.minimum(a, b)
    m2 = jnp.maximum(a, b)
    while h > 128:
        h //= 2
        a1, b1 = m1[:, :h], m1[:, h:]
        a2, b2 = m2[:, :h], m2[:, h:]
        m2 = jnp.minimum(jnp.maximum(a1, b1), jnp.minimum(a2, b2))
        m1 = jnp.minimum(a1, b1)

    a1, a2, a3, a4 = s1_ref[...], s2_ref[...], s3_ref[...], s4_ref[...]
    c1 = jnp.minimum(a1, m1)
    c2 = jnp.minimum(jnp.minimum(a2, jnp.maximum(a1, m1)), m2)
    c3 = jnp.minimum(a3, jnp.minimum(jnp.maximum(a2, m1), jnp.maximum(a1, m2)))
    c4 = jnp.minimum(a4, jnp.minimum(jnp.maximum(a3, m1), jnp.maximum(a2, m2)))
    s1_ref[...] = c1
    s2_ref[...] = c2
    s3_ref[...] = c3
    s4_ref[...] = c4

    @pl.when(j == nsteps - 1)
    def _finish():
        cand = jnp.concatenate([c1, c2, c3, c4], axis=1)
        news = []
        for i in range(_K):
            m = jnp.min(cand, axis=1, keepdims=True)
            news.append(m)
            if i < _K - 1:
                cand = jnp.where(cand == m, inf, cand)
        top5 = jnp.concatenate(news, axis=1)
        bits = jnp.bitwise_and(
            jax.lax.bitcast_convert_type(top5, jnp.int32), jnp.int32(1))
        votes = jnp.sum(bits, axis=1, keepdims=True)
        out_ref[...] = (votes > _K // 2).astype(jnp.float32)


@jax.jit
def kernel(x, X_train, y_train):
    n = X_train.shape[0]
    nc = (n + _C - 1) // _C
    npad = nc * _C - n
    Xp = jnp.pad(X_train, ((0, npad), (0, 0)), constant_values=1e15)
    yp = jnp.pad(y_train, (0, npad)).reshape(nc, 1, _C)
    tqp = jnp.sum(Xp * Xp, axis=1).reshape(nc, 1, _C)
    x2 = x.reshape(_Q, _D) * jnp.float32(-2.0)

    out = pl.pallas_call(
        functools.partial(_knn_body, nsteps=nc),
        grid=(nc,),
        in_specs=[
            pl.BlockSpec((_Q, _D), lambda j: (0, 0)),
            pl.BlockSpec((_C, _D), lambda j: (j, 0)),
            pl.BlockSpec((1, 1, _C), lambda j: (j, 0, 0)),
            pl.BlockSpec((1, 1, _C), lambda j: (j, 0, 0)),
        ],
        out_specs=pl.BlockSpec((_Q, 1), lambda j: (0, 0)),
        out_shape=jax.ShapeDtypeStruct((_Q, 1), jnp.float32),
        scratch_shapes=[pltpu.VMEM((_Q, 128), jnp.float32)] * 4,
    )(x2, Xp, yp, tqp)
    return out


# speedup vs baseline: 1.5484x; 1.1353x over previous
import functools

import jax
import jax.numpy as jnp
from jax.experimental import pallas as pl
from jax.experimental.pallas import tpu as pltpu

_Q = 1024
_D = 16
_K = 5
_C = 4096


def _knn_body(x2_ref, xt_ref, ma_ref, mo_ref, tq_ref, out_ref,
              s1_ref, s2_ref, s3_ref, s4_ref, *, nsteps):
    j = pl.program_id(0)
    inf = jnp.float32(jnp.inf)

    @pl.when(j == 0)
    def _init():
        full = jnp.full((_Q, 128), jnp.inf, dtype=jnp.float32)
        s1_ref[...] = full
        s2_ref[...] = full
        s3_ref[...] = full
        s4_ref[...] = full

    x2 = x2_ref[...]
    xt = xt_ref[...]
    m_and = ma_ref[...]
    m_or = mo_ref[...]
    tq = tq_ref[...]

    cross2 = jax.lax.dot_general(
        x2, xt, dimension_numbers=(((1,), (1,)), ((), ())),
        preferred_element_type=jnp.float32)
    xsq = 0.25 * jnp.sum(x2 * x2, axis=1, keepdims=True)
    d2 = (cross2 + xsq) + tq

    ki = jax.lax.bitcast_convert_type(d2, jnp.int32)
    ki = jnp.bitwise_or(jnp.bitwise_and(ki, m_and), m_or)
    keys = jax.lax.bitcast_convert_type(ki, jnp.float32)

    h = _C // 2
    a, b = keys[:, :h], keys[:, h:]
    m1 = jnp.minimum(a, b)
    m2 = jnp.maximum(a, b)
    while h > 128:
        h //= 2
        a1, b1 = m1[:, :h], m1[:, h:]
        a2, b2 = m2[:, :h], m2[:, h:]
        m2 = jnp.minimum(jnp.maximum(a1, b1), jnp.minimum(a2, b2))
        m1 = jnp.minimum(a1, b1)

    a1, a2, a3, a4 = s1_ref[...], s2_ref[...], s3_ref[...], s4_ref[...]
    c1 = jnp.minimum(a1, m1)
    c2 = jnp.minimum(jnp.minimum(a2, jnp.maximum(a1, m1)), m2)
    c3 = jnp.minimum(a3, jnp.minimum(jnp.maximum(a2, m1), jnp.maximum(a1, m2)))
    c4 = jnp.minimum(a4, jnp.minimum(jnp.maximum(a3, m1), jnp.maximum(a2, m2)))
    s1_ref[...] = c1
    s2_ref[...] = c2
    s3_ref[...] = c3
    s4_ref[...] = c4

    @pl.when(j == nsteps - 1)
    def _finish():
        cand = jnp.concatenate([c1, c2, c3, c4], axis=1)
        news = []
        for i in range(_K):
            m = jnp.min(cand, axis=1, keepdims=True)
            news.append(m)
            if i < _K - 1:
                cand = jnp.where(cand == m, inf, cand)
        top5 = jnp.concatenate(news, axis=1)
        bits = jnp.bitwise_and(
            jax.lax.bitcast_convert_type(top5, jnp.int32), jnp.int32(1))
        votes = jnp.sum(bits, axis=1, keepdims=True)
        out_ref[...] = (votes > _K // 2).astype(jnp.float32)


@jax.jit
def kernel(x, X_train, y_train):
    n = X_train.shape[0]
    nc = (n + _C - 1) // _C
    npad = nc * _C - n
    m_and = jnp.pad(jnp.full((1, n), -2, dtype=jnp.int32), ((0, 0), (0, npad)))
    m_or = jnp.pad(y_train[None, :], ((0, 0), (0, npad)),
                   constant_values=0x7F7FFFFF)
    tqp = jnp.pad(jnp.sum(X_train * X_train, axis=1)[None, :],
                  ((0, 0), (0, npad)))
    x2 = x.reshape(_Q, _D) * jnp.float32(-2.0)

    out = pl.pallas_call(
        functools.partial(_knn_body, nsteps=nc),
        grid=(nc,),
        in_specs=[
            pl.BlockSpec((_Q, _D), lambda j: (0, 0)),
            pl.BlockSpec((_C, _D), lambda j: (j, 0)),
            pl.BlockSpec((1, _C), lambda j: (0, j)),
            pl.BlockSpec((1, _C), lambda j: (0, j)),
            pl.BlockSpec((1, _C), lambda j: (0, j)),
        ],
        out_specs=pl.BlockSpec((_Q, 1), lambda j: (0, 0)),
        out_shape=jax.ShapeDtypeStruct((_Q, 1), jnp.float32),
        scratch_shapes=[pltpu.VMEM((_Q, 128), jnp.float32)] * 4,
    )(x2, X_train, m_and, m_or, tqp)
    return out
